# Initial kernel scaffold; baseline (speedup 1.0000x reference)
import jax
import jax.numpy as jnp
from jax.experimental import pallas as pl


def kernel(query, keys):
    raise NotImplementedError("write your pallas kernel here")


# speedup vs baseline: 63.8835x; 63.8835x over previous
import functools

import jax
import jax.numpy as jnp
from jax import lax
from jax.experimental import pallas as pl
from jax.experimental.pallas import tpu as pltpu
from jax.experimental.pallas import tpu_sc as plsc

Q = 1024
K = 100000
DIM = 128
TOP_K = 5
INV_TEMP = 10.0

B_K = 2048
KP = 102400
N_KB = KP // B_K
T_Q = 128
N_QT = Q // T_Q
CW = 128

NEG_INIT = -6.0
NEG_PAD = -4.0
NEG_MASKED = -5.0
BIG_I32 = 2**31 - 1


def _norm_rows(x):
    n = jnp.sqrt(jnp.sum(x * x, axis=-1, keepdims=True))
    return x / jnp.maximum(n, 1e-12)


def _topk_body(q_ref, k_ref, vals_ref, idx_ref, qn_s, kn_s, vcar_s, icar_s):
    kb = pl.program_id(0)
    qt = pl.program_id(1)

    @pl.when(kb == 0)
    def _init_carry():
        qn_s[qt] = _norm_rows(q_ref[...])
        vcar_s[qt] = jnp.full((T_Q, CW), NEG_INIT, jnp.float32)
        icar_s[qt] = jnp.full((T_Q, CW), BIG_I32, jnp.int32)

    @pl.when(qt == 0)
    def _norm_keys():
        kn_s[...] = _norm_rows(k_ref[...])

    sim = lax.dot_general(
        qn_s[qt], kn_s[...],
        dimension_numbers=(((1,), (1,)), ((), ())),
        preferred_element_type=jnp.float32,
    )

    col = kb * B_K + lax.broadcasted_iota(jnp.int32, (T_Q, B_K), 1)
    sim = jnp.where(col >= K, NEG_PAD, sim)

    av = jnp.concatenate([sim, vcar_s[qt]], axis=1)
    ai = jnp.concatenate([col, icar_s[qt]], axis=1)

    for j in range(TOP_K):
        m = jnp.max(av, axis=1, keepdims=True)
        hit = av >= m
        gi = jnp.min(jnp.where(hit, ai, BIG_I32), axis=1, keepdims=True)
        av = jnp.where(ai == gi, NEG_MASKED, av)
        vcar_s[qt, :, j:j + 1] = m
        icar_s[qt, :, j:j + 1] = gi

    @pl.when(kb == N_KB - 1)
    def _emit():
        vals_ref[...] = vcar_s[qt, :, :8]
        idx_ref[...] = jnp.minimum(icar_s[qt, :, :8], K - 1)


def _run_topk(query, keys_pad):
    return pl.pallas_call(
        _topk_body,
        grid=(N_KB, N_QT),
        in_specs=[
            pl.BlockSpec((T_Q, DIM), lambda kb, qt: (qt, 0)),
            pl.BlockSpec((B_K, DIM), lambda kb, qt: (kb, 0)),
        ],
        out_specs=[
            pl.BlockSpec((T_Q, 8), lambda kb, qt: (qt, 0)),
            pl.BlockSpec((T_Q, 8), lambda kb, qt: (qt, 0)),
        ],
        out_shape=[
            jax.ShapeDtypeStruct((Q, 8), jnp.float32),
            jax.ShapeDtypeStruct((Q, 8), jnp.int32),
        ],
        scratch_shapes=[
            pltpu.VMEM((N_QT, T_Q, DIM), jnp.float32),
            pltpu.VMEM((B_K, DIM), jnp.float32),
            pltpu.VMEM((N_QT, T_Q, CW), jnp.float32),
            pltpu.VMEM((N_QT, T_Q, CW), jnp.int32),
        ],
        compiler_params=pltpu.CompilerParams(
            dimension_semantics=("arbitrary", "arbitrary"),
        ),
    )(query, keys_pad)


_NC = 2
_NS = 16
_NW = _NC * _NS
_NG = Q * TOP_K
_PER_W = _NG // _NW
_CHUNK = 80
_N_CHUNK = _PER_W // _CHUNK


def _sc_gather(keys, idx_flat):
    mesh = plsc.VectorSubcoreMesh(core_axis_name="c", subcore_axis_name="s")

    @functools.partial(
        pl.kernel,
        out_type=jax.ShapeDtypeStruct((_NG, DIM), jnp.float32),
        mesh=mesh,
        scratch_types=[
            pltpu.VMEM((_CHUNK,), jnp.int32),
            pltpu.VMEM((_CHUNK, DIM), jnp.float32),
            pltpu.SemaphoreType.DMA,
        ],
    )
    def gather_k(keys_hbm, idx_hbm, out_hbm, idx_v, rows_v, sem):
        wid = lax.axis_index("s") * _NC + lax.axis_index("c")
        for c in range(_N_CHUNK):
            base = wid * _PER_W + c * _CHUNK
            pltpu.sync_copy(idx_hbm.at[pl.ds(base, _CHUNK)], idx_v)
            pltpu.async_copy(keys_hbm.at[idx_v], rows_v, sem).wait()
            pltpu.sync_copy(rows_v, out_hbm.at[pl.ds(base, _CHUNK)])

    return gather_k(keys, idx_flat)


def _epilogue_body(vals_ref, g_ref, ret_ref, dist_ref, w_ref):
    v = vals_ref[...]
    v0 = v[:, 0:1]
    e = jnp.exp((v - v0) * INV_TEMP)
    w = e / jnp.sum(e, axis=1, keepdims=True)
    w_ref[...] = w
    dist_ref[...] = 1.0 - v0

    g = g_ref[...]
    acc = jnp.zeros((T_Q, DIM), jnp.float32)
    for j in range(TOP_K):
        acc = acc + w[:, j:j + 1] * _norm_rows(g[j])
    ret_ref[...] = _norm_rows(acc)


def _run_epilogue(vals, gathered):
    return pl.pallas_call(
        _epilogue_body,
        grid=(N_QT,),
        in_specs=[
            pl.BlockSpec((T_Q, 8), lambda t: (t, 0)),
            pl.BlockSpec((TOP_K, T_Q, DIM), lambda t: (0, t, 0)),
        ],
        out_specs=[
            pl.BlockSpec((T_Q, DIM), lambda t: (t, 0)),
            pl.BlockSpec((T_Q, 1), lambda t: (t, 0)),
            pl.BlockSpec((T_Q, 8), lambda t: (t, 0)),
        ],
        out_shape=[
            jax.ShapeDtypeStruct((Q, DIM), jnp.float32),
            jax.ShapeDtypeStruct((Q, 1), jnp.float32),
            jax.ShapeDtypeStruct((Q, 8), jnp.float32),
        ],
    )(vals, gathered)


def kernel(query, keys):
    keys_pad = jnp.pad(keys, ((0, KP - K), (0, 0)))
    vals, idx = _run_topk(query, keys_pad)
    idx_flat = jnp.transpose(idx[:, :TOP_K]).reshape(_NG)
    gathered = _sc_gather(keys, idx_flat)
    gathered = gathered.reshape(TOP_K, Q, DIM)
    retrieved, dist, w = _run_epilogue(vals, gathered)
    return retrieved, dist.reshape(Q), w[:, :TOP_K]


# speedup vs baseline: 91.2923x; 1.4290x over previous
import functools

import jax
import jax.numpy as jnp
from jax import lax
from jax.experimental import pallas as pl
from jax.experimental.pallas import tpu as pltpu
from jax.experimental.pallas import tpu_sc as plsc

Q = 1024
K = 100000
DIM = 128
TOP_K = 5
INV_TEMP = 10.0

B_K = 4096
KP = 102400
N_KB = KP // B_K
T_Q = 128
N_QT = Q // T_Q
CW = 8

NEG_INIT = -6.0
NEG_PAD = -4.0
NEG_MASKED = -5.0
BIG_IDX = float(2**24 - 1)
NEG_F32 = -3.0e38


def _norm_rows(x):
    n = jnp.sqrt(jnp.sum(x * x, axis=-1, keepdims=True))
    return x / jnp.maximum(n, 1e-12)


def _topk_body(q_ref, k_ref, vals_ref, idx_ref, qn_s, kn_s, vcar_s, icar_s):
    kb = pl.program_id(0)
    qt = pl.program_id(1)

    @pl.when(kb == 0)
    def _init_carry():
        qn_s[qt] = _norm_rows(q_ref[...])
        vcar_s[qt] = jnp.full((T_Q, CW), NEG_INIT, jnp.float32)
        icar_s[qt] = jnp.full((T_Q, CW), BIG_IDX, jnp.float32)

    @pl.when(qt == 0)
    def _norm_keys():
        kn_s[...] = _norm_rows(k_ref[...])

    sim = lax.dot_general(
        qn_s[qt], kn_s[...],
        dimension_numbers=(((1,), (1,)), ((), ())),
        preferred_element_type=jnp.float32,
    )

    col = (kb * B_K
           + lax.broadcasted_iota(jnp.int32, (T_Q, B_K), 1)).astype(jnp.float32)
    sim = jnp.where(col >= float(K), NEG_PAD, sim)

    av = jnp.concatenate([sim, vcar_s[qt]], axis=1)
    ai = jnp.concatenate([col, icar_s[qt]], axis=1)
    nai = -ai

    for j in range(TOP_K):
        m = jnp.max(av, axis=1, keepdims=True)
        gi = -jnp.max(jnp.where(av >= m, nai, NEG_F32), axis=1, keepdims=True)
        av = jnp.where(ai == gi, NEG_MASKED, av)
        vcar_s[qt, :, j:j + 1] = m
        icar_s[qt, :, j:j + 1] = gi

    @pl.when(kb == N_KB - 1)
    def _emit():
        vals_ref[...] = vcar_s[qt]
        idx_ref[...] = jnp.minimum(icar_s[qt], float(K - 1)).astype(jnp.int32)


def _run_topk(query, keys_pad):
    return pl.pallas_call(
        _topk_body,
        grid=(N_KB, N_QT),
        in_specs=[
            pl.BlockSpec((T_Q, DIM), lambda kb, qt: (qt, 0)),
            pl.BlockSpec((B_K, DIM), lambda kb, qt: (kb, 0)),
        ],
        out_specs=[
            pl.BlockSpec((T_Q, CW), lambda kb, qt: (qt, 0)),
            pl.BlockSpec((T_Q, CW), lambda kb, qt: (qt, 0)),
        ],
        out_shape=[
            jax.ShapeDtypeStruct((Q, CW), jnp.float32),
            jax.ShapeDtypeStruct((Q, CW), jnp.int32),
        ],
        scratch_shapes=[
            pltpu.VMEM((N_QT, T_Q, DIM), jnp.float32),
            pltpu.VMEM((B_K, DIM), jnp.float32),
            pltpu.VMEM((N_QT, T_Q, CW), jnp.float32),
            pltpu.VMEM((N_QT, T_Q, CW), jnp.float32),
        ],
        compiler_params=pltpu.CompilerParams(
            dimension_semantics=("arbitrary", "arbitrary"),
        ),
    )(query, keys_pad)


_NC = 2
_NS = 16
_NW = _NC * _NS
_NG = Q * TOP_K
_PER_W = _NG // _NW
_CHUNK = 80
_N_CHUNK = _PER_W // _CHUNK


def _sc_gather(keys, idx_flat):
    mesh = plsc.VectorSubcoreMesh(core_axis_name="c", subcore_axis_name="s")

    @functools.partial(
        pl.kernel,
        out_type=jax.ShapeDtypeStruct((_NG, DIM), jnp.float32),
        mesh=mesh,
        scratch_types=[
            pltpu.VMEM((_CHUNK,), jnp.int32),
            pltpu.VMEM((_CHUNK, DIM), jnp.float32),
            pltpu.SemaphoreType.DMA,
        ],
    )
    def gather_k(keys_hbm, idx_hbm, out_hbm, idx_v, rows_v, sem):
        wid = lax.axis_index("s") * _NC + lax.axis_index("c")
        for c in range(_N_CHUNK):
            base = wid * _PER_W + c * _CHUNK
            pltpu.sync_copy(idx_hbm.at[pl.ds(base, _CHUNK)], idx_v)
            pltpu.async_copy(keys_hbm.at[idx_v], rows_v, sem).wait()
            pltpu.sync_copy(rows_v, out_hbm.at[pl.ds(base, _CHUNK)])

    return gather_k(keys, idx_flat)


def _epilogue_body(vals_ref, g_ref, ret_ref, dist_ref, w_ref):
    v = vals_ref[...]
    v0 = v[:, 0:1]
    e = jnp.exp((v - v0) * INV_TEMP)
    w = e / jnp.sum(e, axis=1, keepdims=True)
    w_ref[...] = w
    dist_ref[...] = 1.0 - v0

    g = g_ref[...]
    acc = jnp.zeros((T_Q, DIM), jnp.float32)
    for j in range(TOP_K):
        acc = acc + w[:, j:j + 1] * _norm_rows(g[j])
    ret_ref[...] = _norm_rows(acc)


def _run_epilogue(vals, gathered):
    return pl.pallas_call(
        _epilogue_body,
        grid=(N_QT,),
        in_specs=[
            pl.BlockSpec((T_Q, 8), lambda t: (t, 0)),
            pl.BlockSpec((TOP_K, T_Q, DIM), lambda t: (0, t, 0)),
        ],
        out_specs=[
            pl.BlockSpec((T_Q, DIM), lambda t: (t, 0)),
            pl.BlockSpec((T_Q, 1), lambda t: (t, 0)),
            pl.BlockSpec((T_Q, 8), lambda t: (t, 0)),
        ],
        out_shape=[
            jax.ShapeDtypeStruct((Q, DIM), jnp.float32),
            jax.ShapeDtypeStruct((Q, 1), jnp.float32),
            jax.ShapeDtypeStruct((Q, 8), jnp.float32),
        ],
    )(vals, gathered)


def kernel(query, keys):
    keys_pad = jnp.pad(keys, ((0, KP - K), (0, 0)))
    vals, idx = _run_topk(query, keys_pad)
    idx_flat = jnp.transpose(idx[:, :TOP_K]).reshape(_NG)
    gathered = _sc_gather(keys, idx_flat)
    gathered = gathered.reshape(TOP_K, Q, DIM)
    retrieved, dist, w = _run_epilogue(vals, gathered)
    return retrieved, dist.reshape(Q), w[:, :TOP_K]


# speedup vs baseline: 98.1080x; 1.0747x over previous
import functools

import jax
import jax.numpy as jnp
from jax import lax
from jax.experimental import pallas as pl
from jax.experimental.pallas import tpu as pltpu
from jax.experimental.pallas import tpu_sc as plsc

Q = 1024
K = 100000
DIM = 128
TOP_K = 5
INV_TEMP = 10.0

B_K = 4096
KP = 102400
N_KB = KP // B_K
T_Q = 128
N_QT = Q // T_Q
CW = 8
RG = 8

NEG_INIT = -6.0
NEG_PAD = -4.0
NEG_MASKED = -5.0
BIG_IDX = float(2**24 - 1)
NEG_F32 = -3.0e38


def _norm_rows(x):
    n = jnp.sqrt(jnp.sum(x * x, axis=-1, keepdims=True))
    return x / jnp.maximum(n, 1e-12)


def _topk_body(q_ref, k_ref, vals_ref, idx_ref, qn_s, kn_s, vcar_s, icar_s):
    kb = pl.program_id(0)
    qt = pl.program_id(1)

    @pl.when(kb == 0)
    def _init_carry():
        qn_s[qt] = _norm_rows(q_ref[...])
        vcar_s[qt] = jnp.full((T_Q, CW), NEG_INIT, jnp.float32)
        icar_s[qt] = jnp.full((T_Q, CW), BIG_IDX, jnp.float32)

    @pl.when(qt == 0)
    def _norm_keys():
        kn_s[...] = _norm_rows(k_ref[...])

    sim = lax.dot_general(
        qn_s[qt], kn_s[...],
        dimension_numbers=(((1,), (1,)), ((), ())),
        preferred_element_type=jnp.float32,
    )

    col8 = (kb * B_K
            + lax.broadcasted_iota(jnp.int32, (RG, B_K), 1)).astype(jnp.float32)

    for g in range(T_Q // RG):
        rows = slice(g * RG, (g + 1) * RG)
        simg = jnp.where(col8 >= float(K), NEG_PAD, sim[rows, :])
        av = jnp.concatenate([simg, vcar_s[qt, rows, :]], axis=1)
        ai = jnp.concatenate([col8, icar_s[qt, rows, :]], axis=1)
        nai = -ai
        for j in range(TOP_K):
            m = jnp.max(av, axis=1, keepdims=True)
            gi = -jnp.max(jnp.where(av >= m, nai, NEG_F32),
                          axis=1, keepdims=True)
            av = jnp.where(ai == gi, NEG_MASKED, av)
            vcar_s[qt, rows, j:j + 1] = m
            icar_s[qt, rows, j:j + 1] = gi

    @pl.when(kb == N_KB - 1)
    def _emit():
        vals_ref[...] = vcar_s[qt]
        idx_ref[...] = jnp.minimum(icar_s[qt], float(K - 1)).astype(jnp.int32)


def _run_topk(query, keys_pad):
    return pl.pallas_call(
        _topk_body,
        grid=(N_KB, N_QT),
        in_specs=[
            pl.BlockSpec((T_Q, DIM), lambda kb, qt: (qt, 0)),
            pl.BlockSpec((B_K, DIM), lambda kb, qt: (kb, 0)),
        ],
        out_specs=[
            pl.BlockSpec((T_Q, CW), lambda kb, qt: (qt, 0)),
            pl.BlockSpec((T_Q, CW), lambda kb, qt: (qt, 0)),
        ],
        out_shape=[
            jax.ShapeDtypeStruct((Q, CW), jnp.float32),
            jax.ShapeDtypeStruct((Q, CW), jnp.int32),
        ],
        scratch_shapes=[
            pltpu.VMEM((N_QT, T_Q, DIM), jnp.float32),
            pltpu.VMEM((B_K, DIM), jnp.float32),
            pltpu.VMEM((N_QT, T_Q, CW), jnp.float32),
            pltpu.VMEM((N_QT, T_Q, CW), jnp.float32),
        ],
        compiler_params=pltpu.CompilerParams(
            dimension_semantics=("arbitrary", "arbitrary"),
        ),
    )(query, keys_pad)


_NC = 2
_NS = 16
_NW = _NC * _NS
_NG = Q * TOP_K
_PER_W = _NG // _NW
_CHUNK = 80
_N_CHUNK = _PER_W // _CHUNK


def _sc_gather(keys, idx_flat):
    mesh = plsc.VectorSubcoreMesh(core_axis_name="c", subcore_axis_name="s")

    @functools.partial(
        pl.kernel,
        out_type=jax.ShapeDtypeStruct((_NG, DIM), jnp.float32),
        mesh=mesh,
        scratch_types=[
            pltpu.VMEM((_CHUNK,), jnp.int32),
            pltpu.VMEM((_CHUNK, DIM), jnp.float32),
            pltpu.SemaphoreType.DMA,
        ],
    )
    def gather_k(keys_hbm, idx_hbm, out_hbm, idx_v, rows_v, sem):
        wid = lax.axis_index("s") * _NC + lax.axis_index("c")
        for c in range(_N_CHUNK):
            base = wid * _PER_W + c * _CHUNK
            pltpu.sync_copy(idx_hbm.at[pl.ds(base, _CHUNK)], idx_v)
            pltpu.async_copy(keys_hbm.at[idx_v], rows_v, sem).wait()
            pltpu.sync_copy(rows_v, out_hbm.at[pl.ds(base, _CHUNK)])

    return gather_k(keys, idx_flat)


def _epilogue_body(vals_ref, g_ref, ret_ref, dist_ref, w_ref):
    v = vals_ref[...]
    v0 = v[:, 0:1]
    e = jnp.exp((v - v0) * INV_TEMP)
    w = e / jnp.sum(e, axis=1, keepdims=True)
    w_ref[...] = w
    dist_ref[...] = 1.0 - v0

    g = g_ref[...]
    acc = jnp.zeros((T_Q, DIM), jnp.float32)
    for j in range(TOP_K):
        acc = acc + w[:, j:j + 1] * _norm_rows(g[j])
    ret_ref[...] = _norm_rows(acc)


def _run_epilogue(vals, gathered):
    return pl.pallas_call(
        _epilogue_body,
        grid=(N_QT,),
        in_specs=[
            pl.BlockSpec((T_Q, 8), lambda t: (t, 0)),
            pl.BlockSpec((TOP_K, T_Q, DIM), lambda t: (0, t, 0)),
        ],
        out_specs=[
            pl.BlockSpec((T_Q, DIM), lambda t: (t, 0)),
            pl.BlockSpec((T_Q, 1), lambda t: (t, 0)),
            pl.BlockSpec((T_Q, 8), lambda t: (t, 0)),
        ],
        out_shape=[
            jax.ShapeDtypeStruct((Q, DIM), jnp.float32),
            jax.ShapeDtypeStruct((Q, 1), jnp.float32),
            jax.ShapeDtypeStruct((Q, 8), jnp.float32),
        ],
    )(vals, gathered)


def kernel(query, keys):
    keys_pad = jnp.pad(keys, ((0, KP - K), (0, 0)))
    vals, idx = _run_topk(query, keys_pad)
    idx_flat = jnp.transpose(idx[:, :TOP_K]).reshape(_NG)
    gathered = _sc_gather(keys, idx_flat)
    gathered = gathered.reshape(TOP_K, Q, DIM)
    retrieved, dist, w = _run_epilogue(vals, gathered)
    return retrieved, dist.reshape(Q), w[:, :TOP_K]
